# Initial kernel scaffold; baseline (speedup 1.0000x reference)
import functools

import jax
import jax.numpy as jnp
import numpy as np
from jax import lax
from jax.experimental import pallas as pl
from jax.experimental.pallas import tpu as pltpu
from jax.experimental.pallas import tpu_sc as plsc

N = 10000
E = 320000
D = 128
A = 4
B = 8
H = 8

NUM_WORKERS = 32
CH = 128
EDGES_PER_WORKER = 10112
E_PAD = NUM_WORKERS * EDGES_PER_WORKER
N_CHUNKS = EDGES_PER_WORKER // CH
ROWS_PER_SUBCORE = N // 16
STAGE_ROWS = 125

_INV_SQRT_B = 1.0 / np.sqrt(float(B))
_INV_SQRT_H = 1.0 / np.sqrt(float(H))
_INV_SQRT_D = 1.0 / np.sqrt(float(D))
_POST_SCALE = 1.0 / (np.sqrt(32.0) * np.sqrt(float(D)))
_SC_SCALE = 1.0 / np.sqrt(float(D * A))


def _edge_coef_body(emb_ref, ea_ref, w1_ref, w2_ref, out_ref):
    h = jnp.dot(emb_ref[...], w1_ref[...],
                preferred_element_type=jnp.float32) * _INV_SQRT_B
    h = h * jax.nn.sigmoid(h)
    w = jnp.dot(h, w2_ref[...],
                preferred_element_type=jnp.float32) * _INV_SQRT_H
    out_ref[...] = w * ea_ref[...]


def _edge_coef(emb, ea, w1, w2):
    blk = 2048
    grid = E_PAD // blk
    return pl.pallas_call(
        _edge_coef_body,
        grid=(grid,),
        in_specs=[
            pl.BlockSpec((blk, B), lambda i: (i, 0)),
            pl.BlockSpec((blk, 1), lambda i: (i, 0)),
            pl.BlockSpec((B, H), lambda i: (0, 0)),
            pl.BlockSpec((H, D), lambda i: (0, 0)),
        ],
        out_specs=pl.BlockSpec((blk, D), lambda i: (i, 0)),
        out_shape=jax.ShapeDtypeStruct((E_PAD, D), jnp.float32),
    )(emb, ea, w1, w2)


def _lin1_body(x_ref, w_ref, o_ref):
    o_ref[...] = jnp.dot(x_ref[...], w_ref[...],
                         preferred_element_type=jnp.float32) * _INV_SQRT_D


def _lin1(x, w):
    blk = 2000
    return pl.pallas_call(
        _lin1_body,
        grid=(N // blk,),
        in_specs=[
            pl.BlockSpec((blk, D), lambda i: (i, 0)),
            pl.BlockSpec((D, D), lambda i: (0, 0)),
        ],
        out_specs=pl.BlockSpec((blk, D), lambda i: (i, 0)),
        out_shape=jax.ShapeDtypeStruct((N, D), jnp.float32),
    )(x, w)


def _sc_agg_body(src_hbm, dst_hbm, c_hbm, x1_hbm, out_hbm,
                 idx_s, idx_d, c_buf, x_buf, z_buf, acc, sem):
    cid = lax.axis_index("c")
    sid = lax.axis_index("s")
    wid = sid * 2 + cid

    def _zrow(j, carry):
        for k in range(D // 16):
            z_buf[j, pl.ds(k * 16, 16)] = jnp.zeros((16,), jnp.float32)
        return carry
    lax.fori_loop(0, STAGE_ROWS, _zrow, 0)
    for t in range(ROWS_PER_SUBCORE // STAGE_ROWS):
        start = sid * ROWS_PER_SUBCORE + t * STAGE_ROWS
        pltpu.sync_copy(z_buf, acc.at[pl.ds(start, STAGE_ROWS)])
    plsc.subcore_barrier()

    def _chunk(i, carry):
        off = wid * EDGES_PER_WORKER + i * CH
        pltpu.sync_copy(src_hbm.at[pl.ds(off, CH)], idx_s)
        pltpu.sync_copy(dst_hbm.at[pl.ds(off, CH)], idx_d)
        pltpu.sync_copy(c_hbm.at[pl.ds(off, CH)], c_buf)
        pltpu.async_copy(x1_hbm.at[idx_s], x_buf, sem).wait()

        def _row(j, rc):
            for k in range(D // 16):
                sl = pl.ds(k * 16, 16)
                c_buf[j, sl] = c_buf[j, sl] * x_buf[j, sl]
            return rc
        lax.fori_loop(0, CH, _row, 0)
        pltpu.sync_copy(c_buf, acc.at[idx_d], add=True)
        return carry
    lax.fori_loop(0, N_CHUNKS, _chunk, 0)

    plsc.subcore_barrier()
    for t in range(ROWS_PER_SUBCORE // STAGE_ROWS):
        start = sid * ROWS_PER_SUBCORE + t * STAGE_ROWS
        pltpu.sync_copy(acc.at[pl.ds(start, STAGE_ROWS)], z_buf)
        pltpu.sync_copy(z_buf, out_hbm.at[pl.ds(cid * N + start, STAGE_ROWS)])


@functools.partial(
    pl.kernel,
    out_type=jax.ShapeDtypeStruct((2 * N, D), jnp.float32),
    mesh=plsc.VectorSubcoreMesh(core_axis_name="c", subcore_axis_name="s"),
    scratch_types=[
        pltpu.VMEM((CH,), jnp.int32),
        pltpu.VMEM((CH,), jnp.int32),
        pltpu.VMEM((CH, D), jnp.float32),
        pltpu.VMEM((CH, D), jnp.float32),
        pltpu.VMEM((STAGE_ROWS, D), jnp.float32),
        pltpu.VMEM_SHARED((N, D), jnp.float32),
        pltpu.SemaphoreType.DMA,
    ],
)
def _sc_aggregate(src_hbm, dst_hbm, c_hbm, x1_hbm, out_hbm,
                  idx_s, idx_d, c_buf, x_buf, z_buf, acc, sem):
    _sc_agg_body(src_hbm, dst_hbm, c_hbm, x1_hbm, out_hbm,
                 idx_s, idx_d, c_buf, x_buf, z_buf, acc, sem)


def _post_body(p_ref, x_ref, attr_ref, w2_ref, wsc_ref, o_ref):
    agg = p_ref[0] + p_ref[1]
    y = jnp.dot(agg, w2_ref[...],
                preferred_element_type=jnp.float32) * _POST_SCALE
    for v in range(A):
        y = y + jnp.dot(x_ref[...] * attr_ref[:, v:v + 1], wsc_ref[v],
                        preferred_element_type=jnp.float32) * _SC_SCALE
    o_ref[...] = y


def _post(partial, x, attr, w2, wsc_t):
    blk = 2000
    return pl.pallas_call(
        _post_body,
        grid=(N // blk,),
        in_specs=[
            pl.BlockSpec((2, blk, D), lambda i: (0, i, 0)),
            pl.BlockSpec((blk, D), lambda i: (i, 0)),
            pl.BlockSpec((blk, A), lambda i: (i, 0)),
            pl.BlockSpec((D, D), lambda i: (0, 0)),
            pl.BlockSpec((A, D, D), lambda i: (0, 0, 0)),
        ],
        out_specs=pl.BlockSpec((blk, D), lambda i: (i, 0)),
        out_shape=jax.ShapeDtypeStruct((N, D), jnp.float32),
    )(partial, x, attr, w2, wsc_t)


def kernel(node_features, node_attr, edge_attr, edge_embedding, edge_index,
           W_lin1, fc_W1, fc_W2, W_lin2, W_sc):
    pad = E_PAD - E
    src_p = jnp.pad(edge_index[0], (0, pad))
    dst_p = jnp.pad(edge_index[1], (0, pad))
    emb_p = jnp.pad(edge_embedding, ((0, pad), (0, 0)))
    ea_p = jnp.pad(edge_attr, ((0, pad), (0, 0)))

    c = _edge_coef(emb_p, ea_p, fc_W1, fc_W2)
    x1 = _lin1(node_features, W_lin1)
    partial = _sc_aggregate(src_p, dst_p, c, x1)
    return _post(partial.reshape(2, N, D), node_features, node_attr,
                 W_lin2, W_sc.transpose(1, 0, 2))


# speedup vs baseline: 1.7078x; 1.7078x over previous
import functools

import jax
import jax.numpy as jnp
import numpy as np
from jax import lax
from jax.experimental import pallas as pl
from jax.experimental.pallas import tpu as pltpu
from jax.experimental.pallas import tpu_sc as plsc

N = 10000
E = 320000
D = 128
A = 4
B = 8
H = 8

NUM_WORKERS = 32
CH = 128
EDGES_PER_WORKER = 10112
E_PAD = NUM_WORKERS * EDGES_PER_WORKER
N_CHUNKS = EDGES_PER_WORKER // CH
N_PAD = 10240
ROWS_PER_SUBCORE = N_PAD // 16
STAGE_ROWS = 128

_INV_SQRT_B = 1.0 / np.sqrt(float(B))
_INV_SQRT_H = 1.0 / np.sqrt(float(H))
_INV_SQRT_D = 1.0 / np.sqrt(float(D))
_POST_SCALE = 1.0 / (np.sqrt(32.0) * np.sqrt(float(D)))
_SC_SCALE = 1.0 / np.sqrt(float(D * A))


def _edge_coef_body(emb_ref, ea_ref, w1_ref, w2_ref, out_ref):
    h = jnp.dot(emb_ref[...], w1_ref[...],
                preferred_element_type=jnp.float32) * _INV_SQRT_B
    h = h * jax.nn.sigmoid(h)
    w = jnp.dot(h, w2_ref[...],
                preferred_element_type=jnp.float32) * _INV_SQRT_H
    out_ref[...] = w * ea_ref[...]


def _edge_coef(emb, ea, w1, w2):
    blk = 2048
    grid = E_PAD // blk
    return pl.pallas_call(
        _edge_coef_body,
        grid=(grid,),
        in_specs=[
            pl.BlockSpec((blk, B), lambda i: (i, 0)),
            pl.BlockSpec((blk, 1), lambda i: (i, 0)),
            pl.BlockSpec((B, H), lambda i: (0, 0)),
            pl.BlockSpec((H, D), lambda i: (0, 0)),
        ],
        out_specs=pl.BlockSpec((blk, D), lambda i: (i, 0)),
        out_shape=jax.ShapeDtypeStruct((E_PAD, D), jnp.float32),
    )(emb, ea, w1, w2)


def _lin1_body(x_ref, w_ref, o_ref):
    o_ref[...] = jnp.dot(x_ref[...], w_ref[...],
                         preferred_element_type=jnp.float32) * _INV_SQRT_D


def _lin1(x, w):
    blk = 2000
    return pl.pallas_call(
        _lin1_body,
        grid=(N // blk,),
        in_specs=[
            pl.BlockSpec((blk, D), lambda i: (i, 0)),
            pl.BlockSpec((D, D), lambda i: (0, 0)),
        ],
        out_specs=pl.BlockSpec((blk, D), lambda i: (i, 0)),
        out_shape=jax.ShapeDtypeStruct((N, D), jnp.float32),
    )(x, w)


def _sc_agg_body(src_hbm, dst_hbm, c_hbm, x1_hbm, out_hbm,
                 idx_s, idx_d, c_buf, x_buf, acc, sem):
    cid = lax.axis_index("c")
    sid = lax.axis_index("s")
    wid = sid * 2 + cid

    def _zrow(j, carry):
        for k in range(D // 16):
            c_buf[j, pl.ds(k * 16, 16)] = jnp.zeros((16,), jnp.float32)
        return carry
    lax.fori_loop(0, STAGE_ROWS, _zrow, 0)
    for t in range(ROWS_PER_SUBCORE // STAGE_ROWS):
        start = sid * ROWS_PER_SUBCORE + t * STAGE_ROWS
        pltpu.sync_copy(c_buf, acc.at[pl.ds(start, STAGE_ROWS)])
    plsc.subcore_barrier()

    def _chunk(i, carry):
        off = wid * EDGES_PER_WORKER + i * CH
        pltpu.sync_copy(src_hbm.at[pl.ds(off, CH)], idx_s)
        pltpu.sync_copy(dst_hbm.at[pl.ds(off, CH)], idx_d)
        pltpu.sync_copy(c_hbm.at[pl.ds(off, CH)], c_buf)
        pltpu.async_copy(x1_hbm.at[idx_s], x_buf, sem).wait()

        def _row(j, rc):
            for k in range(D // 16):
                sl = pl.ds(k * 16, 16)
                c_buf[j, sl] = c_buf[j, sl] * x_buf[j, sl]
            return rc
        lax.fori_loop(0, CH, _row, 0)
        pltpu.sync_copy(c_buf, acc.at[idx_d], add=True)
        return carry
    lax.fori_loop(0, N_CHUNKS, _chunk, 0)

    plsc.subcore_barrier()
    for t in range(ROWS_PER_SUBCORE // STAGE_ROWS):
        start = sid * ROWS_PER_SUBCORE + t * STAGE_ROWS
        pltpu.sync_copy(acc.at[pl.ds(start, STAGE_ROWS)], c_buf)
        pltpu.sync_copy(c_buf, out_hbm.at[pl.ds(cid * N_PAD + start, STAGE_ROWS)])


@functools.partial(
    pl.kernel,
    out_type=jax.ShapeDtypeStruct((2 * N_PAD, D), jnp.float32),
    mesh=plsc.VectorSubcoreMesh(core_axis_name="c", subcore_axis_name="s"),
    scratch_types=[
        pltpu.VMEM((CH,), jnp.int32),
        pltpu.VMEM((CH,), jnp.int32),
        pltpu.VMEM((CH, D), jnp.float32),
        pltpu.VMEM((CH, D), jnp.float32),
        pltpu.VMEM_SHARED((N_PAD, D), jnp.float32),
        pltpu.SemaphoreType.DMA,
    ],
)
def _sc_aggregate(src_hbm, dst_hbm, c_hbm, x1_hbm, out_hbm,
                  idx_s, idx_d, c_buf, x_buf, acc, sem):
    _sc_agg_body(src_hbm, dst_hbm, c_hbm, x1_hbm, out_hbm,
                 idx_s, idx_d, c_buf, x_buf, acc, sem)


def _post_body(p_ref, x_ref, attr_ref, w2_ref, wsc_ref, o_ref):
    agg = p_ref[0] + p_ref[1]
    y = jnp.dot(agg, w2_ref[...],
                preferred_element_type=jnp.float32) * _POST_SCALE
    for v in range(A):
        y = y + jnp.dot(x_ref[...] * attr_ref[:, v:v + 1], wsc_ref[v],
                        preferred_element_type=jnp.float32) * _SC_SCALE
    o_ref[...] = y


def _post(partial, x, attr, w2, wsc_t):
    blk = 2000
    return pl.pallas_call(
        _post_body,
        grid=(N // blk,),
        in_specs=[
            pl.BlockSpec((2, blk, D), lambda i: (0, i, 0)),
            pl.BlockSpec((blk, D), lambda i: (i, 0)),
            pl.BlockSpec((blk, A), lambda i: (i, 0)),
            pl.BlockSpec((D, D), lambda i: (0, 0)),
            pl.BlockSpec((A, D, D), lambda i: (0, 0, 0)),
        ],
        out_specs=pl.BlockSpec((blk, D), lambda i: (i, 0)),
        out_shape=jax.ShapeDtypeStruct((N, D), jnp.float32),
    )(partial, x, attr, w2, wsc_t)


def kernel(node_features, node_attr, edge_attr, edge_embedding, edge_index,
           W_lin1, fc_W1, fc_W2, W_lin2, W_sc):
    pad = E_PAD - E
    src_p = jnp.pad(edge_index[0], (0, pad))
    dst_p = jnp.pad(edge_index[1], (0, pad))
    emb_p = jnp.pad(edge_embedding, ((0, pad), (0, 0)))
    ea_p = jnp.pad(edge_attr, ((0, pad), (0, 0)))

    c = _edge_coef(emb_p, ea_p, fc_W1, fc_W2)
    x1 = _lin1(node_features, W_lin1)
    partial = _sc_aggregate(src_p, dst_p, c, x1)
    partial = partial.reshape(2, N_PAD, D)[:, :N, :]
    return _post(partial, node_features, node_attr,
                 W_lin2, W_sc.transpose(1, 0, 2))


# speedup vs baseline: 2.2223x; 1.3013x over previous
import functools

import jax
import jax.numpy as jnp
import numpy as np
from jax import lax
from jax.experimental import pallas as pl
from jax.experimental.pallas import tpu as pltpu
from jax.experimental.pallas import tpu_sc as plsc

N = 10000
E = 320000
D = 128
A = 4
B = 8
H = 8

NUM_WORKERS = 32
CH = 88
N_CHUNKS = 114
EDGES_PER_WORKER = CH * N_CHUNKS
E_PAD = NUM_WORKERS * EDGES_PER_WORKER
N_PAD = 10240
ROWS_PER_SUBCORE = N_PAD // 16
STAGE_ROWS = 80

_INV_SQRT_B = 1.0 / np.sqrt(float(B))
_INV_SQRT_H = 1.0 / np.sqrt(float(H))
_INV_SQRT_D = 1.0 / np.sqrt(float(D))
_POST_SCALE = 1.0 / (np.sqrt(32.0) * np.sqrt(float(D)))
_SC_SCALE = 1.0 / np.sqrt(float(D * A))


def _edge_coef_body(emb_ref, ea_ref, w1_ref, w2_ref, out_ref):
    h = jnp.dot(emb_ref[...], w1_ref[...],
                preferred_element_type=jnp.float32) * _INV_SQRT_B
    h = h * jax.nn.sigmoid(h)
    w = jnp.dot(h, w2_ref[...],
                preferred_element_type=jnp.float32) * _INV_SQRT_H
    out_ref[...] = w * ea_ref[...]


def _edge_coef(emb, ea, w1, w2):
    blk = 1408
    grid = E_PAD // blk
    return pl.pallas_call(
        _edge_coef_body,
        grid=(grid,),
        in_specs=[
            pl.BlockSpec((blk, B), lambda i: (i, 0)),
            pl.BlockSpec((blk, 1), lambda i: (i, 0)),
            pl.BlockSpec((B, H), lambda i: (0, 0)),
            pl.BlockSpec((H, D), lambda i: (0, 0)),
        ],
        out_specs=pl.BlockSpec((blk, D), lambda i: (i, 0)),
        out_shape=jax.ShapeDtypeStruct((E_PAD, D), jnp.float32),
    )(emb, ea, w1, w2)


def _lin1_body(x_ref, w_ref, o_ref):
    o_ref[...] = jnp.dot(x_ref[...], w_ref[...],
                         preferred_element_type=jnp.float32) * _INV_SQRT_D


def _lin1(x, w):
    blk = 2000
    return pl.pallas_call(
        _lin1_body,
        grid=(N // blk,),
        in_specs=[
            pl.BlockSpec((blk, D), lambda i: (i, 0)),
            pl.BlockSpec((D, D), lambda i: (0, 0)),
        ],
        out_specs=pl.BlockSpec((blk, D), lambda i: (i, 0)),
        out_shape=jax.ShapeDtypeStruct((N, D), jnp.float32),
    )(x, w)


def _sc_agg_body(idx_hbm, c_hbm, x1_hbm, out_hbm,
                 idx0, idx1, c0, c1, x0, x1b,
                 acc, sem_c0, sem_c1, sem_x0, sem_x1):
    cid = lax.axis_index("c")
    sid = lax.axis_index("s")
    wid = sid * 2 + cid
    chunk0 = wid * N_CHUNKS

    def _zrow(j, carry):
        for k in range(D // 16):
            c0[j, pl.ds(k * 16, 16)] = jnp.zeros((16,), jnp.float32)
        return carry
    lax.fori_loop(0, STAGE_ROWS, _zrow, 0)
    for t in range(ROWS_PER_SUBCORE // STAGE_ROWS):
        start = sid * ROWS_PER_SUBCORE + t * STAGE_ROWS
        pltpu.sync_copy(c0.at[pl.ds(0, STAGE_ROWS)], acc.at[pl.ds(start, STAGE_ROWS)])
    plsc.subcore_barrier()

    def _prefetch(i, idx_b, c_b, x_b, sem_c, sem_x):
        pltpu.sync_copy(idx_hbm.at[chunk0 + i], idx_b)
        pltpu.async_copy(c_hbm.at[pl.ds((chunk0 + i) * CH, CH)], c_b, sem_c)
        pltpu.async_copy(x1_hbm.at[idx_b.at[0]], x_b, sem_x)

    def _consume(idx_b, c_b, x_b, sem_c, sem_x):
        pltpu.make_async_copy(c_hbm.at[pl.ds(0, CH)], c_b, sem_c).wait()
        pltpu.make_async_copy(x1_hbm.at[idx_b.at[0]], x_b, sem_x).wait()

        def _row(j, rc):
            for k in range(D // 16):
                sl = pl.ds(k * 16, 16)
                c_b[j, sl] = c_b[j, sl] * x_b[j, sl]
            return rc
        lax.fori_loop(0, CH, _row, 0)
        pltpu.sync_copy(c_b, acc.at[idx_b.at[1]], add=True)

    _prefetch(0, idx0, c0, x0, sem_c0, sem_x0)

    def _pair(t, carry):
        i = 2 * t
        _prefetch(i + 1, idx1, c1, x1b, sem_c1, sem_x1)
        _consume(idx0, c0, x0, sem_c0, sem_x0)

        @pl.when(i + 2 < N_CHUNKS)
        def _():
            _prefetch(i + 2, idx0, c0, x0, sem_c0, sem_x0)
        _consume(idx1, c1, x1b, sem_c1, sem_x1)
        return carry
    lax.fori_loop(0, N_CHUNKS // 2, _pair, 0)

    plsc.subcore_barrier()
    for t in range(ROWS_PER_SUBCORE // STAGE_ROWS):
        start = sid * ROWS_PER_SUBCORE + t * STAGE_ROWS
        pltpu.sync_copy(acc.at[pl.ds(start, STAGE_ROWS)], c0.at[pl.ds(0, STAGE_ROWS)])
        pltpu.sync_copy(c0.at[pl.ds(0, STAGE_ROWS)],
                        out_hbm.at[pl.ds(cid * N_PAD + start, STAGE_ROWS)])


@functools.partial(
    pl.kernel,
    out_type=jax.ShapeDtypeStruct((2 * N_PAD, D), jnp.float32),
    mesh=plsc.VectorSubcoreMesh(core_axis_name="c", subcore_axis_name="s"),
    scratch_types=[
        pltpu.VMEM((2, CH), jnp.int32),
        pltpu.VMEM((2, CH), jnp.int32),
        pltpu.VMEM((CH, D), jnp.float32),
        pltpu.VMEM((CH, D), jnp.float32),
        pltpu.VMEM((CH, D), jnp.float32),
        pltpu.VMEM((CH, D), jnp.float32),
        pltpu.VMEM_SHARED((N_PAD, D), jnp.float32),
        pltpu.SemaphoreType.DMA,
        pltpu.SemaphoreType.DMA,
        pltpu.SemaphoreType.DMA,
        pltpu.SemaphoreType.DMA,
    ],
)
def _sc_aggregate(idx_hbm, c_hbm, x1_hbm, out_hbm,
                  idx0, idx1, c0, c1, x0, x1b,
                  acc, sem_c0, sem_c1, sem_x0, sem_x1):
    _sc_agg_body(idx_hbm, c_hbm, x1_hbm, out_hbm,
                 idx0, idx1, c0, c1, x0, x1b,
                 acc, sem_c0, sem_c1, sem_x0, sem_x1)


def _post_body(p_ref, x_ref, attr_ref, w2_ref, wsc_ref, o_ref):
    agg = p_ref[0] + p_ref[1]
    y = jnp.dot(agg, w2_ref[...],
                preferred_element_type=jnp.float32) * _POST_SCALE
    for v in range(A):
        y = y + jnp.dot(x_ref[...] * attr_ref[:, v:v + 1], wsc_ref[v],
                        preferred_element_type=jnp.float32) * _SC_SCALE
    o_ref[...] = y


def _post(partial, x, attr, w2, wsc_t):
    blk = 2000
    return pl.pallas_call(
        _post_body,
        grid=(N // blk,),
        in_specs=[
            pl.BlockSpec((2, blk, D), lambda i: (0, i, 0)),
            pl.BlockSpec((blk, D), lambda i: (i, 0)),
            pl.BlockSpec((blk, A), lambda i: (i, 0)),
            pl.BlockSpec((D, D), lambda i: (0, 0)),
            pl.BlockSpec((A, D, D), lambda i: (0, 0, 0)),
        ],
        out_specs=pl.BlockSpec((blk, D), lambda i: (i, 0)),
        out_shape=jax.ShapeDtypeStruct((N, D), jnp.float32),
    )(partial, x, attr, w2, wsc_t)


def kernel(node_features, node_attr, edge_attr, edge_embedding, edge_index,
           W_lin1, fc_W1, fc_W2, W_lin2, W_sc):
    pad = E_PAD - E
    src_p = jnp.pad(edge_index[0], (0, pad))
    dst_p = jnp.pad(edge_index[1], (0, pad))
    emb_p = jnp.pad(edge_embedding, ((0, pad), (0, 0)))
    ea_p = jnp.pad(edge_attr, ((0, pad), (0, 0)))
    idx_packed = jnp.stack(
        [src_p.reshape(-1, CH), dst_p.reshape(-1, CH)], axis=1)

    c = _edge_coef(emb_p, ea_p, fc_W1, fc_W2)
    x1 = _lin1(node_features, W_lin1)
    partial = _sc_aggregate(idx_packed, c, x1)
    partial = partial.reshape(2, N_PAD, D)[:, :N, :]
    return _post(partial, node_features, node_attr,
                 W_lin2, W_sc.transpose(1, 0, 2))
